# baseline (speedup 1.0000x reference)
import dataclasses
import functools

import jax
import jax.numpy as jnp
from jax import lax
from jax.experimental import pallas as pl
from jax.experimental.pallas import tpu as pltpu
from jax.experimental.pallas import tpu_sc as plsc

N_NODES = 10000
N_EDGES = 320000
D_FEAT = 128
NC = 2
NS = 16
L = 16
NW = NC * NS
E_PER_W = N_EDGES // NW
GW = 128
NWIN = E_PER_W // GW
TAIL = E_PER_W - NWIN * GW
D32 = D_FEAT // 2
NBUF = 3


@jax.jit
def kernel(h, edge_index):
    ei = edge_index.astype(jnp.int32)
    u1 = ei[0]
    v1 = ei[1]
    hb = h.astype(jnp.bfloat16)
    h32 = lax.bitcast_convert_type(hb.reshape(N_NODES, D32, 2), jnp.uint32)

    mesh = plsc.VectorSubcoreMesh(core_axis_name="c", subcore_axis_name="s")
    cp = pltpu.CompilerParams()
    for _f, _v in (("needs_layout_passes", False),
                   ("use_tc_tiling_on_sc", False)):
        if _f in pltpu.CompilerParams.__dataclass_fields__:
            cp = dataclasses.replace(cp, **{_f: _v})

    @functools.partial(
        pl.kernel,
        out_type=jax.ShapeDtypeStruct((N_EDGES,), jnp.float32),
        mesh=mesh,
        compiler_params=cp,
        scratch_types=[
            pltpu.VMEM((E_PER_W,), jnp.int32),
            pltpu.VMEM((E_PER_W,), jnp.int32),
            pltpu.VMEM((NBUF, GW, D32), jnp.uint32),
            pltpu.VMEM((NBUF, GW, D32), jnp.uint32),
            pltpu.VMEM((E_PER_W,), jnp.float32),
        ] + [pltpu.SemaphoreType.DMA] * (2 * NBUF),
    )
    def k(h_hbm, u_hbm, v_hbm, out_hbm,
          idx_u, idx_v, ring_u, ring_v, out_v, *sems):
        wid = lax.axis_index("s") * NC + lax.axis_index("c")
        base = wid * E_PER_W
        pltpu.sync_copy(u_hbm.at[pl.ds(base, E_PER_W)], idx_u)
        pltpu.sync_copy(v_hbm.at[pl.ds(base, E_PER_W)], idx_v)

        bufs_u = tuple(ring_u.at[b] for b in range(NBUF))
        bufs_v = tuple(ring_v.at[b] for b in range(NBUF))
        sems_u = sems[:NBUF]
        sems_v = sems[NBUF:]

        def start(jj, b, n):
            pltpu.async_copy(h_hbm.at[idx_u.at[pl.ds(jj * GW, n)]],
                             bufs_u[b].at[pl.ds(0, n)], sems_u[b])
            pltpu.async_copy(h_hbm.at[idx_v.at[pl.ds(jj * GW, n)]],
                             bufs_v[b].at[pl.ds(0, n)], sems_v[b])

        def wait(b, n):
            pltpu.make_async_copy(h_hbm.at[pl.ds(0, n), :],
                                  bufs_u[b].at[pl.ds(0, n)], sems_u[b]).wait()
            pltpu.make_async_copy(h_hbm.at[pl.ds(0, n), :],
                                  bufs_v[b].at[pl.ds(0, n)], sems_v[b]).wait()

        hi_mask = jnp.full((L,), 0xFFFF0000, jnp.uint32)
        shift16 = jnp.full((L,), 16, jnp.uint32)

        def dot_terms(ru, rv, i, c):
            wu = ru[i, pl.ds(c * L, L)]
            wv = rv[i, pl.ds(c * L, L)]
            pu = plsc.bitcast(wu, jnp.bfloat16)
            pv = plsc.bitcast(wv, jnp.bfloat16)
            pw = plsc.bitcast(pu * pv, jnp.uint32)
            lo = plsc.bitcast(lax.shift_left(pw, shift16), jnp.float32)
            hi = plsc.bitcast(pw & hi_mask, jnp.float32)
            return lo + hi

        lane = jax.lax.iota(jnp.int32, L)
        _dn = lax.GatherDimensionNumbers(
            offset_dims=(), collapsed_slice_dims=(0,), start_index_map=(0,))

        def perm(x, idx):
            return lax.gather(x, idx[:, None], _dn, slice_sizes=(1,),
                              mode=lax.GatherScatterMode.PROMISE_IN_BOUNDS)

        def lane_merge(x, y, k):
            return jnp.where((lane & k) == 0,
                             x + perm(x, lane ^ k),
                             y + perm(y, lane ^ k))

        def block16(ru, rv, i0, out_off):
            accs = [dot_terms(ru, rv, i0 + r, 0) for r in range(L)]
            for c in range(1, D32 // L):
                for r in range(L):
                    accs[r] = accs[r] + dot_terms(ru, rv, i0 + r, c)
            vecs = accs
            k = 1
            while len(vecs) > 1:
                vecs = [lane_merge(vecs[i], vecs[i + 1], k)
                        for i in range(0, len(vecs), 2)]
                k *= 2
            out_v[pl.ds(out_off, L)] = 1.0 / (1.0 + jnp.exp(-vecs[0]))

        def compute(jj, ru, rv):
            @pl.loop(0, GW, step=L)
            def _(i0):
                block16(ru, rv, i0, jj * GW + i0)

        for b in range(NBUF):
            start(b, b, GW)

        @pl.loop(0, NWIN, step=NBUF)
        def _(j):
            for b in range(NBUF):
                jj = j + b
                wait(b, GW)
                compute(jj, bufs_u[b], bufs_v[b])

                @pl.when(jj + NBUF < NWIN)
                def _():
                    start(jj + NBUF, b, GW)

        start(NWIN, 0, TAIL)
        wait(0, TAIL)
        block16(bufs_u[0], bufs_v[0], 0, NWIN * GW)

        pltpu.sync_copy(out_v, out_hbm.at[pl.ds(base, E_PER_W)])

    return k(h32, u1, v1)


# speedup vs baseline: 1.1152x; 1.1152x over previous
import dataclasses
import functools

import jax
import jax.numpy as jnp
from jax import lax
from jax.experimental import pallas as pl
from jax.experimental.pallas import tpu as pltpu
from jax.experimental.pallas import tpu_sc as plsc

N_NODES = 10000
N_EDGES = 320000
D_FEAT = 128
NC = 2
NS = 16
L = 16
NW = NC * NS
E_PER_W = N_EDGES // NW
GW = 128
NWIN = E_PER_W // GW
TAIL = E_PER_W - NWIN * GW
D32 = D_FEAT // 2
NBUF = 2


@jax.jit
def kernel(h, edge_index):
    ei = edge_index.astype(jnp.int32)
    u1 = ei[0]
    v1 = ei[1]
    hb = h.astype(jnp.bfloat16)
    h32 = lax.bitcast_convert_type(hb.reshape(N_NODES, D32, 2), jnp.uint32)

    mesh = plsc.VectorSubcoreMesh(core_axis_name="c", subcore_axis_name="s")
    cp = pltpu.CompilerParams()
    for _f, _v in (("needs_layout_passes", False),
                   ("use_tc_tiling_on_sc", False)):
        if _f in pltpu.CompilerParams.__dataclass_fields__:
            cp = dataclasses.replace(cp, **{_f: _v})

    @functools.partial(
        pl.kernel,
        out_type=jax.ShapeDtypeStruct((N_EDGES,), jnp.float32),
        mesh=mesh,
        compiler_params=cp,
        scratch_types=[
            pltpu.VMEM((E_PER_W,), jnp.int32),
            pltpu.VMEM((E_PER_W,), jnp.int32),
            pltpu.VMEM((NBUF, GW, D32), jnp.uint32),
            pltpu.VMEM((NBUF, GW, D32), jnp.uint32),
            pltpu.VMEM((E_PER_W,), jnp.float32),
        ] + [pltpu.SemaphoreType.DMA((NBUF,)),
             pltpu.SemaphoreType.DMA((NBUF,))],
    )
    def k(h_hbm, u_hbm, v_hbm, out_hbm,
          idx_u, idx_v, ring_u, ring_v, out_v, sem_u, sem_v):
        wid = lax.axis_index("s") * NC + lax.axis_index("c")
        base = wid * E_PER_W
        pltpu.sync_copy(u_hbm.at[pl.ds(base, E_PER_W)], idx_u)
        pltpu.sync_copy(v_hbm.at[pl.ds(base, E_PER_W)], idx_v)

        def start(jj, b, n):
            pltpu.async_copy(h_hbm.at[idx_u.at[pl.ds(jj * GW, n)]],
                             ring_u.at[b].at[pl.ds(0, n)], sem_u.at[b])
            pltpu.async_copy(h_hbm.at[idx_v.at[pl.ds(jj * GW, n)]],
                             ring_v.at[b].at[pl.ds(0, n)], sem_v.at[b])

        def wait(b, n):
            pltpu.make_async_copy(h_hbm.at[pl.ds(0, n), :],
                                  ring_u.at[b].at[pl.ds(0, n)], sem_u.at[b]).wait()
            pltpu.make_async_copy(h_hbm.at[pl.ds(0, n), :],
                                  ring_v.at[b].at[pl.ds(0, n)], sem_v.at[b]).wait()

        hi_mask = jnp.full((L,), 0xFFFF0000, jnp.uint32)
        shift16 = jnp.full((L,), 16, jnp.uint32)

        def dot_terms(ru, rv, i, c):
            wu = ru[i, pl.ds(c * L, L)]
            wv = rv[i, pl.ds(c * L, L)]
            pu = plsc.bitcast(wu, jnp.bfloat16)
            pv = plsc.bitcast(wv, jnp.bfloat16)
            pw = plsc.bitcast(pu * pv, jnp.uint32)
            lo = plsc.bitcast(lax.shift_left(pw, shift16), jnp.float32)
            hi = plsc.bitcast(pw & hi_mask, jnp.float32)
            return lo + hi

        lane = jax.lax.iota(jnp.int32, L)
        _dn = lax.GatherDimensionNumbers(
            offset_dims=(), collapsed_slice_dims=(0,), start_index_map=(0,))

        def perm(x, idx):
            return lax.gather(x, idx[:, None], _dn, slice_sizes=(1,),
                              mode=lax.GatherScatterMode.PROMISE_IN_BOUNDS)

        def lane_merge(x, y, k):
            return jnp.where((lane & k) == 0,
                             x + perm(x, lane ^ k),
                             y + perm(y, lane ^ k))

        def block16(ru, rv, i0, out_off):
            accs = [dot_terms(ru, rv, i0 + r, 0) for r in range(L)]
            for c in range(1, D32 // L):
                for r in range(L):
                    accs[r] = accs[r] + dot_terms(ru, rv, i0 + r, c)
            vecs = accs
            k = 1
            while len(vecs) > 1:
                vecs = [lane_merge(vecs[i], vecs[i + 1], k)
                        for i in range(0, len(vecs), 2)]
                k *= 2
            out_v[pl.ds(out_off, L)] = 1.0 / (1.0 + jnp.exp(-vecs[0]))

        def compute(jj, ru, rv):
            @pl.loop(0, GW, step=L)
            def _(i0):
                block16(ru, rv, i0, jj * GW + i0)

        for b in range(NBUF):
            start(b, b, GW)

        @pl.loop(0, NWIN)
        def _(j):
            b = j & (NBUF - 1)
            wait(b, GW)
            compute(j, ring_u.at[b], ring_v.at[b])

            @pl.when(j + NBUF < NWIN)
            def _():
                start(j + NBUF, b, GW)

        start(NWIN, 0, TAIL)
        wait(0, TAIL)
        block16(ring_u.at[0], ring_v.at[0], 0, NWIN * GW)

        pltpu.sync_copy(out_v, out_hbm.at[pl.ds(base, E_PER_W)])

    return k(h32, u1, v1)
